# Initial kernel scaffold; baseline (speedup 1.0000x reference)
import jax
import jax.numpy as jnp
import numpy as np
from jax import lax
from jax.experimental import pallas as pl
from jax.experimental.pallas import tpu as pltpu
from jax.experimental.pallas import tpu_sc as plsc

D = 64
L = 200
CB = 4
NV = D // 16


def _cbow_body(ids_hbm, table_hbm, out_hbm, idx_v, rows_v, ostag, sem):
    nc = 2
    ns = 16
    wid = lax.axis_index("c") * ns + lax.axis_index("s")
    b_total = out_hbm.shape[0]
    rows_per_w = b_total // (nc * ns)
    n_chunks = rows_per_w // CB
    row0 = wid * rows_per_w
    inv_l = np.float32(1.0 / L)

    @pl.loop(0, n_chunks)
    def _chunk(g):
        r0 = row0 + g * CB
        pltpu.sync_copy(ids_hbm.at[pl.ds(r0 * L, CB * L)], idx_v)
        pltpu.async_copy(table_hbm.at[idx_v], rows_v, sem).wait()

        for cb in range(CB):
            zero = jnp.zeros((16,), jnp.float32)

            @pl.loop(0, L, init_carry=(zero,) * NV, unroll=8)
            def _acc(l, carry):
                r = cb * L + l
                return tuple(
                    carry[d] + rows_v[r, pl.ds(d * 16, 16)] for d in range(NV)
                )

            for d in range(NV):
                ostag[cb, pl.ds(d * 16, 16)] = _acc[d] * inv_l

        pltpu.sync_copy(ostag, out_hbm.at[pl.ds(r0, CB)])


@jax.jit
def kernel(input_ids, table):
    b, l = input_ids.shape
    assert l == L and table.shape[1] == D
    ids_flat = input_ids.reshape(-1).astype(jnp.int32)

    mesh = plsc.VectorSubcoreMesh(core_axis_name="c", subcore_axis_name="s")
    k = pl.kernel(
        _cbow_body,
        out_type=jax.ShapeDtypeStruct((b, D), jnp.float32),
        mesh=mesh,
        scratch_types=[
            pltpu.VMEM((CB * L,), jnp.int32),
            pltpu.VMEM((CB * L, D), jnp.float32),
            pltpu.VMEM((CB, D), jnp.float32),
            pltpu.SemaphoreType.DMA,
        ],
    )
    return k(ids_flat, table)


# speedup vs baseline: 2.4740x; 2.4740x over previous
import jax
import jax.numpy as jnp
import numpy as np
from jax import lax
from jax.experimental import pallas as pl
from jax.experimental.pallas import tpu as pltpu
from jax.experimental.pallas import tpu_sc as plsc

D = 64
L = 200
CB = 4
NV = D // 16


def _cbow_body(ids_hbm, table_hbm, out_hbm, idx_v, rows_v, ostag, sem):
    nc = 2
    ns = 16
    wid = lax.axis_index("c") * ns + lax.axis_index("s")
    b_total = out_hbm.shape[0]
    rows_per_w = b_total // (nc * ns)
    n_chunks = rows_per_w // CB
    row0 = wid * rows_per_w
    inv_l = np.float32(1.0 / L)

    @pl.loop(0, n_chunks)
    def _chunk(g):
        r0 = row0 + g * CB
        pltpu.sync_copy(ids_hbm.at[pl.ds(r0 * L, CB * L)], idx_v)
        pltpu.async_copy(table_hbm.at[idx_v], rows_v, sem).wait()

        for cb in range(CB):
            zero = jnp.zeros((16,), jnp.float32)

            @pl.loop(0, L, init_carry=(zero,) * NV, unroll=8)
            def _acc(l, carry):
                r = cb * L + l
                return tuple(
                    carry[d] + rows_v[r, pl.ds(d * 16, 16)] for d in range(NV)
                )

            for d in range(NV):
                ostag[cb, pl.ds(d * 16, 16)] = _acc[d] * inv_l

        pltpu.sync_copy(ostag, out_hbm.at[pl.ds(r0, CB)])


@jax.jit
def kernel(input_ids, table):
    b, l = input_ids.shape
    assert l == L and table.shape[1] == D
    ids_flat = input_ids.reshape(-1).astype(jnp.int32)

    mesh = plsc.VectorSubcoreMesh(core_axis_name="c", subcore_axis_name="s")
    k = pl.kernel(
        _cbow_body,
        out_type=jax.ShapeDtypeStruct((b, D), jnp.float32),
        mesh=mesh,
        scratch_types=[
            pltpu.VMEM((CB * L,), jnp.int32),
            pltpu.VMEM((CB * L, D), jnp.float32),
            pltpu.VMEM((CB, D), jnp.float32),
            pltpu.SemaphoreType.DMA,
        ],
        compiler_params=pltpu.CompilerParams(use_tc_tiling_on_sc=False),
    )
    return k(ids_flat, table)


# speedup vs baseline: 3.2374x; 1.3086x over previous
import jax
import jax.numpy as jnp
import numpy as np
from jax import lax
from jax.experimental import pallas as pl
from jax.experimental.pallas import tpu as pltpu
from jax.experimental.pallas import tpu_sc as plsc

D = 64
L = 200
CB = 4
NV = D // 16


OB = 8


def _cbow_body(ids_hbm, table_hbm, out_hbm,
               idx0, idx1, rows0, rows1, ostag, sem0, sem1):
    nc = 2
    ns = 16
    wid = lax.axis_index("c") * ns + lax.axis_index("s")
    b_total = out_hbm.shape[0]
    rows_per_w = b_total // (nc * ns)
    n_chunks = rows_per_w // CB
    row0 = wid * rows_per_w
    inv_l = np.float32(1.0 / L)
    bufs = ((idx0, rows0, sem0), (idx1, rows1, sem1))

    def load_idx_and_fire(g, idxb, rowsb, semb):
        r0 = row0 + g * CB
        pltpu.sync_copy(ids_hbm.at[pl.ds(r0 * L, CB * L)], idxb)
        pltpu.async_copy(table_hbm.at[idxb], rowsb, semb)

    load_idx_and_fire(0, *bufs[0])

    @pl.loop(0, n_chunks, step=2)
    def _chunk2(g2):
        for b in (0, 1):
            g = g2 + b
            idxb, rowsb, semb = bufs[b]

            @pl.when(g + 1 < n_chunks)
            def _fire_next():
                load_idx_and_fire(g + 1, *bufs[1 - b])

            pltpu.make_async_copy(table_hbm.at[idxb], rowsb, semb).wait()

            pos = g % OB
            for cb in range(CB):
                zero = jnp.zeros((16,), jnp.float32)

                @pl.loop(0, L, init_carry=(zero,) * NV, unroll=8)
                def _acc(l, carry):
                    r = cb * L + l
                    return tuple(
                        carry[d] + rowsb[r, pl.ds(d * 16, 16)]
                        for d in range(NV)
                    )

                for d in range(NV):
                    ostag[pos * CB + cb, pl.ds(d * 16, 16)] = _acc[d] * inv_l

            @pl.when(pos == OB - 1)
            def _flush_out():
                pltpu.sync_copy(
                    ostag,
                    out_hbm.at[pl.ds(row0 + (g + 1 - OB) * CB, OB * CB)],
                )


@jax.jit
def kernel(input_ids, table):
    b, l = input_ids.shape
    assert l == L and table.shape[1] == D
    ids_flat = input_ids.reshape(-1).astype(jnp.int32)

    mesh = plsc.VectorSubcoreMesh(core_axis_name="c", subcore_axis_name="s")
    k = pl.kernel(
        _cbow_body,
        out_type=jax.ShapeDtypeStruct((b, D), jnp.float32),
        mesh=mesh,
        scratch_types=[
            pltpu.VMEM((CB * L,), jnp.int32),
            pltpu.VMEM((CB * L,), jnp.int32),
            pltpu.VMEM((CB * L, D), jnp.float32),
            pltpu.VMEM((CB * L, D), jnp.float32),
            pltpu.VMEM((OB * CB, D), jnp.float32),
            pltpu.SemaphoreType.DMA,
            pltpu.SemaphoreType.DMA,
        ],
        compiler_params=pltpu.CompilerParams(use_tc_tiling_on_sc=False),
    )
    return k(ids_flat, table)
